# baseline (speedup 1.0000x reference)
import functools

import jax
import jax.numpy as jnp
from jax import lax
from jax.experimental import pallas as pl
from jax.experimental.pallas import tpu as pltpu
from jax.experimental.pallas import tpu_sc as plsc

HIDDEN = 64
WIDE = 128
NC = 2
NS = 16
NW = NC * NS
CHUNK = 128
NUM_TOK = 1000000
BANDS = NUM_TOK // WIDE
TAIL = NUM_TOK - BANDS * WIDE


GRP = 3
GBANDS = BANDS // GRP
GW = GRP * WIDE
GR = GRP * HIDDEN


def _transpose_call():
    mesh = plsc.VectorSubcoreMesh(core_axis_name="c", subcore_axis_name="s")
    n_t = GBANDS // NW + 2
    if n_t % 2:
        n_t += 1

    @functools.partial(
        pl.kernel,
        mesh=mesh,
        compiler_params=pltpu.CompilerParams(needs_layout_passes=False),
        out_type=jax.ShapeDtypeStruct((NUM_TOK // 2, WIDE), jnp.float32),
        scratch_types=[
            [pltpu.VMEM((HIDDEN, GW), jnp.float32)] * 2,
            [pltpu.VMEM((GR, WIDE), jnp.float32)] * 2,
            pltpu.VMEM((HIDDEN, TAIL), jnp.float32),
            [pltpu.SemaphoreType.DMA] * 4,
        ],
    )
    def k(tokT_hbm, t2_hbm, tbufs, obufs, tailb, sems):
        wid = lax.axis_index("s") * NC + lax.axis_index("c")
        i_sem = sems[:2]
        o_sem = sems[2:]
        lane = lax.iota(jnp.int32, 16)
        row_l = lax.shift_right_logical(lane, 1)
        col_l = lax.bitwise_and(lane, 1) * HIDDEN

        def group_of(t):
            return wid + NW * t

        def issue_in(s, t):
            off = pl.multiple_of(group_of(t) * GW, GW)
            pltpu.async_copy(tokT_hbm.at[:, pl.ds(off, GW)],
                             tbufs[s], i_sem[s])

        def drain_in(s):
            pltpu.make_async_copy(tokT_hbm.at[:, pl.ds(0, GW)],
                                  tbufs[s], i_sem[s]).wait()

        def drain_out(s):
            pltpu.make_async_copy(obufs[s], t2_hbm.at[pl.ds(0, GR)],
                                  o_sem[s]).wait()

        rows_km = tuple(tuple(row_l + HIDDEN * kb + 8 * m for m in range(8))
                        for kb in range(GRP))

        def transpose(src, dst):
            @plsc.parallel_loop(0, HIDDEN, unroll=2)
            def _col_body(c):
                colv = col_l + c
                for kb in range(GRP):
                    for m in range(8):
                        val = src[c, pl.ds(WIDE * kb + 16 * m, 16)]
                        plsc.store_scatter(dst, [rows_km[kb][m], colv], val)

        def store_out(s, t):
            off = pl.multiple_of(group_of(t) * GR, GR)
            pltpu.async_copy(obufs[s], t2_hbm.at[pl.ds(off, GR)],
                             o_sem[s])

        @pl.when(group_of(0) < GBANDS)
        def _():
            issue_in(0, 0)

        @pl.when(group_of(1) < GBANDS)
        def _():
            issue_in(1, 1)

        def pair_body(kk, carry):
            for s in range(2):
                t = 2 * kk + s
                active = group_of(t) < GBANDS

                @pl.when(active)
                def _():
                    drain_in(s)

                    @pl.when(t >= 2)
                    def _():
                        drain_out(s)

                    transpose(tbufs[s], obufs[s])
                    store_out(s, t)

                    @pl.when(group_of(t + 2) < GBANDS)
                    def _():
                        issue_in(s, t + 2)
            return carry

        lax.fori_loop(0, n_t // 2, pair_body, 0)
        drain_out(0)
        drain_out(1)

        @pl.when(wid == 0)
        def _():
            pltpu.sync_copy(tokT_hbm.at[:, pl.ds(BANDS * WIDE, TAIL)], tailb)

            @plsc.parallel_loop(0, HIDDEN, unroll=4)
            def _tail_body(c):
                colv = col_l + c
                for m in range(TAIL // 16):
                    val = tailb[c, pl.ds(16 * m, 16)]
                    plsc.store_scatter(obufs[0], [rows_km[0][m], colv], val)

            pltpu.sync_copy(obufs[0].at[pl.ds(0, TAIL // 2)],
                            t2_hbm.at[pl.ds(BANDS * HIDDEN, TAIL // 2)])

    return k


def _gather_call(N):
    n_w = N // NW
    n_chunks = n_w // CHUNK
    n_pairs = n_chunks // 2
    mesh = plsc.VectorSubcoreMesh(core_axis_name="c", subcore_axis_name="s")

    row_buf = pltpu.VMEM((CHUNK, WIDE), jnp.float32)

    @functools.partial(
        pl.kernel,
        mesh=mesh,
        compiler_params=pltpu.CompilerParams(needs_layout_passes=False),
        out_type=jax.ShapeDtypeStruct((N // 2, WIDE), jnp.float32),
        scratch_types=[
            pltpu.VMEM((n_w,), jnp.int32),
            pltpu.VMEM((n_w,), jnp.int32),
            pltpu.VMEM((n_w,), jnp.int32),
            pltpu.VMEM((n_w,), jnp.int32),
            pltpu.VMEM((WIDE,), jnp.float32),
            [row_buf] * 3,
            [row_buf] * 3,
            [pltpu.SemaphoreType.DMA] * 4,
        ],
    )
    def k(tok_i_hbm, pos_i_hbm, turn_i_hbm,
          t2_hbm, pos_t_hbm, turn_t_hbm, te_hbm,
          out_hbm,
          tok_idx, tokp_idx, pos_idx, turn_idx, te_v, slot_a, slot_b, sems):
        wid = lax.axis_index("s") * NC + lax.axis_index("c")
        base = pl.multiple_of(wid * n_w, n_w)
        pltpu.sync_copy(tok_i_hbm.at[pl.ds(base, n_w)], tok_idx)
        pltpu.sync_copy(pos_i_hbm.at[pl.ds(base, n_w)], pos_idx)
        pltpu.sync_copy(turn_i_hbm.at[pl.ds(base, n_w)], turn_idx)
        pltpu.sync_copy(te_hbm, te_v)

        @plsc.parallel_loop(0, n_w // 16)
        def _half_body(v):
            sl = pl.ds(v * 16, 16)
            tokp_idx[sl] = lax.shift_right_logical(tok_idx[sl], 1)

        g_sem = sems[:2]
        s_sem = sems[2:]
        slots = (slot_a, slot_b)

        def issue3(s, g):
            tokv, posv, turnv = slots[s]
            off = pl.multiple_of(g * CHUNK, CHUNK)
            pltpu.async_copy(t2_hbm.at[tokp_idx.at[pl.ds(off, CHUNK)]],
                             tokv, g_sem[s])
            pltpu.async_copy(pos_t_hbm.at[pos_idx.at[pl.ds(off, CHUNK)]],
                             posv, g_sem[s])
            pltpu.async_copy(turn_t_hbm.at[turn_idx.at[pl.ds(off, CHUNK)]],
                             turnv, g_sem[s])

        def drain_gathers(s):
            for buf in slots[s]:
                pltpu.make_async_copy(out_hbm.at[pl.ds(0, CHUNK)],
                                      buf, g_sem[s]).wait()

        def drain_store(s):
            pltpu.make_async_copy(
                slots[s][0].at[pl.ds(0, CHUNK // 2)],
                out_hbm.at[pl.ds(0, CHUNK // 2)], s_sem[s]).wait()

        def compute(s, g):
            tokv, posv, turnv = slots[s]
            te = tuple(te_v[pl.ds(j * 16, 16)] for j in range(HIDDEN // 16))
            off0 = g * CHUNK

            def _blk_body(b, te_c):
                i0 = b * 16
                parvec = lax.bitwise_and(
                    tok_idx[pl.ds(off0 + i0, 16)], 1) * HIDDEN
                for l in range(16):
                    i = i0 + l
                    p64 = parvec[l]
                    m = 8 * b + l // 2
                    for j in range(HIDDEN // 16):
                        tokcol = tokv[i, pl.ds(p64 + 16 * j, 16)]
                        res = (tokcol + posv[i, pl.ds(16 * j, 16)]
                               + turnv[i, pl.ds(16 * j, 16)] + te_c[j])
                        tokv[m, pl.ds((l % 2) * HIDDEN + 16 * j, 16)] = res
                return te_c

            lax.fori_loop(0, CHUNK // 16, _blk_body, te)

        def store(s, g):
            off = pl.multiple_of((base + g * CHUNK) // 2, CHUNK // 2)
            pltpu.async_copy(
                slots[s][0].at[pl.ds(0, CHUNK // 2)],
                out_hbm.at[pl.ds(off, CHUNK // 2)],
                s_sem[s])

        issue3(0, 0)
        issue3(1, 1)

        def pair_body(kk, carry):
            for s in range(2):
                g = 2 * kk + s
                drain_gathers(s)
                compute(s, g)
                store(s, g)

                @pl.when(kk < n_pairs - 1)
                def _():
                    drain_store(s)
                    issue3(s, g + 2)
            return carry

        lax.fori_loop(0, n_pairs, pair_body, 0)
        drain_store(0)
        drain_store(1)

    return k


@functools.lru_cache(maxsize=None)
def _build(N):
    return _transpose_call(), _gather_call(N)


def kernel(token_inp, pos_inp, turn_inp, token_table, pos_table, turn_table,
           text_embedding):
    B, L = token_inp.shape
    N = B * L
    t_call, g_call = _build(N)
    t2 = t_call(token_table.T)
    pad = ((0, 0), (0, WIDE - HIDDEN))
    out2 = g_call(
        token_inp.reshape(N), pos_inp.reshape(N), turn_inp.reshape(N),
        t2, jnp.pad(pos_table, pad), jnp.pad(turn_table, pad),
        jnp.pad(text_embedding, (0, WIDE - HIDDEN)))
    return out2.reshape(-1).reshape(B, L, HIDDEN)


# speedup vs baseline: 1.2946x; 1.2946x over previous
import functools

import jax
import jax.numpy as jnp
from jax import lax
from jax.experimental import pallas as pl
from jax.experimental.pallas import tpu as pltpu
from jax.experimental.pallas import tpu_sc as plsc

HIDDEN = 64
NC = 2
NS = 16
NW = NC * NS
CHUNK = 128


def _posturn_call(N):
    n_w = N // NW
    n_pairs = (n_w // CHUNK) // 2
    mesh = plsc.VectorSubcoreMesh(core_axis_name="c", subcore_axis_name="s")
    row_buf = pltpu.VMEM((CHUNK, HIDDEN), jnp.float32)

    @functools.partial(
        pl.kernel,
        mesh=mesh,
        compiler_params=pltpu.CompilerParams(use_tc_tiling_on_sc=False),
        out_type=jax.ShapeDtypeStruct((N, HIDDEN), jnp.float32),
        scratch_types=[
            pltpu.VMEM((n_w,), jnp.int32),
            pltpu.VMEM((n_w,), jnp.int32),
            pltpu.VMEM((HIDDEN,), jnp.float32),
            [row_buf] * 2,
            [row_buf] * 2,
            [pltpu.SemaphoreType.DMA] * 4,
        ],
    )
    def k(pos_i_hbm, turn_i_hbm, pos_t_hbm, turn_t_hbm, te_hbm,
          out_hbm, pos_idx, turn_idx, te_v, slot_a, slot_b, sems):
        wid = lax.axis_index("s") * NC + lax.axis_index("c")
        base = wid * n_w
        pltpu.sync_copy(pos_i_hbm.at[pl.ds(base, n_w)], pos_idx)
        pltpu.sync_copy(turn_i_hbm.at[pl.ds(base, n_w)], turn_idx)
        pltpu.sync_copy(te_hbm, te_v)
        g_sem, s_sem = sems[:2], sems[2:]
        slots = (slot_a, slot_b)

        def issue2(s, g):
            posv, turnv = slots[s]
            off = g * CHUNK
            pltpu.async_copy(pos_t_hbm.at[pos_idx.at[pl.ds(off, CHUNK)]],
                             posv, g_sem[s])
            pltpu.async_copy(turn_t_hbm.at[turn_idx.at[pl.ds(off, CHUNK)]],
                             turnv, g_sem[s])

        def drain_gathers(s):
            for buf in slots[s]:
                pltpu.make_async_copy(out_hbm.at[pl.ds(0, CHUNK)],
                                      buf, g_sem[s]).wait()

        def drain_store(s):
            pltpu.make_async_copy(slots[s][0], out_hbm.at[pl.ds(0, CHUNK)],
                                  s_sem[s]).wait()

        def compute(s):
            posv, turnv = slots[s]
            te = tuple(te_v[pl.ds(j * 16, 16)] for j in range(HIDDEN // 16))

            @plsc.parallel_loop(0, CHUNK, unroll=8, carry=te)
            def _row_body(i, te_c):
                for j in range(HIDDEN // 16):
                    sl = pl.ds(j * 16, 16)
                    posv[i, sl] = posv[i, sl] + turnv[i, sl] + te_c[j]
                return te_c

        def store(s, g):
            pltpu.async_copy(slots[s][0],
                             out_hbm.at[pl.ds(base + g * CHUNK, CHUNK)],
                             s_sem[s])

        issue2(0, 0)
        issue2(1, 1)

        def pair_body(kk, carry):
            for s in range(2):
                g = 2 * kk + s
                drain_gathers(s)
                compute(s)
                store(s, g)

                @pl.when(kk < n_pairs - 1)
                def _():
                    drain_store(s)
                    issue2(s, g + 2)
            return carry

        lax.fori_loop(0, n_pairs, pair_body, 0)
        drain_store(0)
        drain_store(1)

    return k


def _token_call(N):
    n_w = N // NW
    n_pairs = (n_w // CHUNK) // 2
    mesh = plsc.VectorSubcoreMesh(core_axis_name="c", subcore_axis_name="s")
    row_buf = pltpu.VMEM((CHUNK, HIDDEN), jnp.float32)

    @functools.partial(
        pl.kernel,
        mesh=mesh,
        compiler_params=pltpu.CompilerParams(use_tc_tiling_on_sc=False),
        out_type=jax.ShapeDtypeStruct((N, HIDDEN), jnp.float32),
        scratch_types=[
            pltpu.VMEM((n_w,), jnp.int32),
            [row_buf] * 2,
            [row_buf] * 2,
            [pltpu.SemaphoreType.DMA] * 4,
        ],
    )
    def k(tok_i_hbm, tok_t_hbm, pt_hbm,
          out_hbm, tok_idx, slot_a, slot_b, sems):
        wid = lax.axis_index("s") * NC + lax.axis_index("c")
        base = wid * n_w
        pltpu.sync_copy(tok_i_hbm.at[pl.ds(base, n_w)], tok_idx)
        g_sem, s_sem = sems[:2], sems[2:]
        slots = (slot_a, slot_b)

        def issue2(s, g):
            tokv, ptv = slots[s]
            off = g * CHUNK
            pltpu.async_copy(tok_t_hbm.at[tok_idx.at[pl.ds(off, CHUNK)]],
                             tokv, g_sem[s])
            pltpu.async_copy(pt_hbm.at[pl.ds(base + off, CHUNK)],
                             ptv, g_sem[s])

        def drain_gathers(s):
            for buf in slots[s]:
                pltpu.make_async_copy(out_hbm.at[pl.ds(0, CHUNK)],
                                      buf, g_sem[s]).wait()

        def drain_store(s):
            pltpu.make_async_copy(slots[s][0], out_hbm.at[pl.ds(0, CHUNK)],
                                  s_sem[s]).wait()

        def compute(s):
            tokv, ptv = slots[s]

            @plsc.parallel_loop(0, CHUNK, unroll=8)
            def _row_body(i):
                for j in range(HIDDEN // 16):
                    sl = pl.ds(j * 16, 16)
                    tokv[i, sl] = tokv[i, sl] + ptv[i, sl]

        def store(s, g):
            pltpu.async_copy(slots[s][0],
                             out_hbm.at[pl.ds(base + g * CHUNK, CHUNK)],
                             s_sem[s])

        issue2(0, 0)
        issue2(1, 1)

        def pair_body(kk, carry):
            for s in range(2):
                g = 2 * kk + s
                drain_gathers(s)
                compute(s)
                store(s, g)

                @pl.when(kk < n_pairs - 1)
                def _():
                    drain_store(s)
                    issue2(s, g + 2)
            return carry

        lax.fori_loop(0, n_pairs, pair_body, 0)
        drain_store(0)
        drain_store(1)

    return k


@functools.lru_cache(maxsize=None)
def _build(N):
    return _posturn_call(N), _token_call(N)


def kernel(token_inp, pos_inp, turn_inp, token_table, pos_table, turn_table,
           text_embedding):
    B, L = token_inp.shape
    N = B * L
    pt_call, tok_call = _build(N)
    posturn = pt_call(pos_inp.reshape(N), turn_inp.reshape(N),
                      pos_table, turn_table, text_embedding)
    out = tok_call(token_inp.reshape(N), token_table, posturn)
    return out.reshape(B, L, HIDDEN)


# speedup vs baseline: 1.3029x; 1.0064x over previous
import functools

import jax
import jax.numpy as jnp
from jax import lax
from jax.experimental import pallas as pl
from jax.experimental.pallas import tpu as pltpu
from jax.experimental.pallas import tpu_sc as plsc

HIDDEN = 64
NC = 2
NS = 16
NW = NC * NS
CHUNK = 128


def _posturn_call(N):
    n_w = N // NW
    n_pairs = (n_w // CHUNK) // 2
    mesh = plsc.VectorSubcoreMesh(core_axis_name="c", subcore_axis_name="s")
    row_buf = pltpu.VMEM((CHUNK, HIDDEN), jnp.float32)

    @functools.partial(
        pl.kernel,
        mesh=mesh,
        compiler_params=pltpu.CompilerParams(use_tc_tiling_on_sc=False),
        out_type=jax.ShapeDtypeStruct((N, HIDDEN), jnp.float32),
        scratch_types=[
            pltpu.VMEM((n_w,), jnp.int32),
            pltpu.VMEM((n_w,), jnp.int32),
            pltpu.VMEM((HIDDEN,), jnp.float32),
            [row_buf] * 2,
            [row_buf] * 2,
            [pltpu.SemaphoreType.DMA] * 4,
        ],
    )
    def k(pos_i_hbm, turn_i_hbm, pos_t_hbm, turn_t_hbm, te_hbm,
          out_hbm, pos_idx, turn_idx, te_v, slot_a, slot_b, sems):
        wid = lax.axis_index("s") * NC + lax.axis_index("c")
        base = wid * n_w
        pltpu.sync_copy(pos_i_hbm.at[pl.ds(base, n_w)], pos_idx)
        pltpu.sync_copy(turn_i_hbm.at[pl.ds(base, n_w)], turn_idx)
        pltpu.sync_copy(te_hbm, te_v)
        g_sem, s_sem = sems[:2], sems[2:]
        slots = (slot_a, slot_b)

        def issue2(s, g):
            posv, turnv = slots[s]
            off = g * CHUNK
            pltpu.async_copy(pos_t_hbm.at[pos_idx.at[pl.ds(off, CHUNK)]],
                             posv, g_sem[s])
            pltpu.async_copy(turn_t_hbm.at[turn_idx.at[pl.ds(off, CHUNK)]],
                             turnv, g_sem[s])

        def drain_gathers(s):
            for buf in slots[s]:
                pltpu.make_async_copy(out_hbm.at[pl.ds(0, CHUNK)],
                                      buf, g_sem[s]).wait()

        def drain_store(s):
            pltpu.make_async_copy(slots[s][0], out_hbm.at[pl.ds(0, CHUNK)],
                                  s_sem[s]).wait()

        def compute(s):
            posv, turnv = slots[s]
            te = tuple(te_v[pl.ds(j * 16, 16)] for j in range(HIDDEN // 16))

            @plsc.parallel_loop(0, CHUNK, unroll=8, carry=te)
            def _row_body(i, te_c):
                for j in range(HIDDEN // 16):
                    sl = pl.ds(j * 16, 16)
                    posv[i, sl] = posv[i, sl] + turnv[i, sl] + te_c[j]
                return te_c

        def store(s, g):
            pltpu.async_copy(slots[s][0],
                             out_hbm.at[pl.ds(base + g * CHUNK, CHUNK)],
                             s_sem[s])

        issue2(0, 0)
        issue2(1, 1)

        def pair_body(kk, carry):
            for s in range(2):
                g = 2 * kk + s
                drain_gathers(s)
                compute(s)
                store(s, g)

                @pl.when(kk < n_pairs - 1)
                def _():
                    drain_store(s)
                    issue2(s, g + 2)
            return carry

        lax.fori_loop(0, n_pairs, pair_body, 0)
        drain_store(0)
        drain_store(1)

    return k


def _token_call(N, B, L):
    n_w = N // NW
    nb_w = n_w // L
    n_pairs = nb_w // 2
    c0 = 128
    c1 = L - c0
    mesh = plsc.VectorSubcoreMesh(core_axis_name="c", subcore_axis_name="s")
    row_buf = pltpu.VMEM((L, HIDDEN), jnp.float32)

    @functools.partial(
        pl.kernel,
        mesh=mesh,
        compiler_params=pltpu.CompilerParams(use_tc_tiling_on_sc=False),
        out_type=jax.ShapeDtypeStruct((B, L, HIDDEN), jnp.float32),
        scratch_types=[
            pltpu.VMEM((n_w,), jnp.int32),
            [row_buf] * 2,
            [row_buf] * 2,
            [pltpu.SemaphoreType.DMA] * 4,
        ],
    )
    def k(tok_i_hbm, tok_t_hbm, pt_hbm,
          out_hbm, tok_idx, slot_a, slot_b, sems):
        wid = lax.axis_index("s") * NC + lax.axis_index("c")
        base = pl.multiple_of(wid * n_w, n_w)
        pltpu.sync_copy(tok_i_hbm.at[pl.ds(base, n_w)], tok_idx)
        g_sem, s_sem = sems[:2], sems[2:]
        slots = (slot_a, slot_b)

        def issue(s, g):
            tokv, ptv = slots[s]
            off = pl.multiple_of(g * L, L)
            pltpu.async_copy(
                tok_t_hbm.at[tok_idx.at[pl.ds(off, c0)]],
                tokv.at[pl.ds(0, c0)], g_sem[s])
            pltpu.async_copy(
                tok_t_hbm.at[tok_idx.at[pl.ds(off + c0, c1)]],
                tokv.at[pl.ds(c0, c1)], g_sem[s])
            pltpu.async_copy(pt_hbm.at[pl.ds(base + off, L)], ptv, g_sem[s])

        def drain_gathers(s):
            tokv, ptv = slots[s]
            pltpu.make_async_copy(pt_hbm.at[pl.ds(0, c0)],
                                  tokv.at[pl.ds(0, c0)], g_sem[s]).wait()
            pltpu.make_async_copy(pt_hbm.at[pl.ds(0, c1)],
                                  tokv.at[pl.ds(0, c1)], g_sem[s]).wait()
            pltpu.make_async_copy(pt_hbm.at[pl.ds(0, L)], ptv,
                                  g_sem[s]).wait()

        def drain_store(s):
            pltpu.make_async_copy(slots[s][0], out_hbm.at[0],
                                  s_sem[s]).wait()

        def compute(s):
            tokv, ptv = slots[s]

            @plsc.parallel_loop(0, L, unroll=8)
            def _row_body(i):
                for j in range(HIDDEN // 16):
                    sl = pl.ds(j * 16, 16)
                    tokv[i, sl] = tokv[i, sl] + ptv[i, sl]

        def store(s, g):
            bb = wid * nb_w + g
            pltpu.async_copy(slots[s][0], out_hbm.at[bb], s_sem[s])

        issue(0, 0)
        issue(1, 1)

        def pair_body(kk, carry):
            for s in range(2):
                g = 2 * kk + s
                drain_gathers(s)
                compute(s)
                store(s, g)

                @pl.when(kk < n_pairs - 1)
                def _():
                    drain_store(s)
                    issue(s, g + 2)
            return carry

        lax.fori_loop(0, n_pairs, pair_body, 0)
        drain_store(0)
        drain_store(1)

    return k


@functools.lru_cache(maxsize=None)
def _build(N, B, L):
    return _posturn_call(N), _token_call(N, B, L)


def kernel(token_inp, pos_inp, turn_inp, token_table, pos_table, turn_table,
           text_embedding):
    B, L = token_inp.shape
    N = B * L
    pt_call, tok_call = _build(N, B, L)
    posturn = pt_call(pos_inp.reshape(N), turn_inp.reshape(N),
                      pos_table, turn_table, text_embedding)
    return tok_call(token_inp.reshape(N), token_table, posturn)
